# Initial kernel scaffold; baseline (speedup 1.0000x reference)
import jax
import jax.numpy as jnp
from jax.experimental import pallas as pl


def kernel(tokens, lengths):
    raise NotImplementedError("write your pallas kernel here")


# speedup vs baseline: 1.7741x; 1.7741x over previous
import functools

import jax
import jax.numpy as jnp
from jax import lax
from jax.experimental import pallas as pl
from jax.experimental.pallas import tpu as pltpu
from jax.experimental.pallas import tpu_sc as plsc

VOCAB = 100000
BATCH = 1024
SEQ = 200
N_WORKERS = 32
ROWS_PER_WORKER = BATCH // N_WORKERS
LANES = 16
FULL_GROUPS = SEQ // LANES
TAIL = SEQ - FULL_GROUPS * LANES


def _sc_body(tokens_hbm, out_hbm, tok_v, hist_v):
    wid = lax.axis_index("s") * 2 + lax.axis_index("c")
    base = wid * ROWS_PER_WORKER

    pltpu.sync_copy(tokens_hbm.at[pl.ds(base, ROWS_PER_WORKER)], tok_v)

    lane = lax.iota(jnp.int32, LANES)
    zero_f = jnp.zeros((LANES,), jnp.float32)

    def zero_body(i, c):
        hist_v[pl.ds(i * LANES, LANES)] = zero_f
        return c

    lax.fori_loop(0, VOCAB // LANES, zero_body, 0)

    def scatter_row(r, value):
        val = jnp.full((LANES,), value, jnp.float32)
        for g in range(FULL_GROUPS):
            tv = tok_v[r, pl.ds(g * LANES, LANES)]
            for j in range(LANES):
                m = (lane == j) & (tv != 0)
                plsc.addupdate_scatter(hist_v, [tv], val, mask=m)
        tv = tok_v[r, pl.ds(SEQ - LANES, LANES)]
        for j in range(LANES - TAIL, LANES):
            m = (lane == j) & (tv != 0)
            plsc.addupdate_scatter(hist_v, [tv], val, mask=m)

    def row_body(r, c):
        scatter_row(r, 1.0)
        pltpu.sync_copy(hist_v, out_hbm.at[base + r])
        scatter_row(r, -1.0)
        return c

    lax.fori_loop(0, ROWS_PER_WORKER, row_body, 0)


@jax.jit
def _encode(tokens):
    mesh = plsc.VectorSubcoreMesh(core_axis_name="c", subcore_axis_name="s")
    return pl.kernel(
        _sc_body,
        out_type=jax.ShapeDtypeStruct((BATCH, VOCAB), jnp.float32),
        mesh=mesh,
        compiler_params=pltpu.CompilerParams(needs_layout_passes=False),
        scratch_types=[
            pltpu.VMEM((ROWS_PER_WORKER, SEQ), jnp.int32),
            pltpu.VMEM((VOCAB,), jnp.float32),
        ],
    )(tokens)


def kernel(tokens, lengths):
    del lengths
    return _encode(tokens)
